# Initial kernel scaffold; baseline (speedup 1.0000x reference)
import jax
import jax.numpy as jnp
from jax.experimental import pallas as pl


def kernel(pred, target):
    raise NotImplementedError("write your pallas kernel here")


# speedup vs baseline: 79.0852x; 79.0852x over previous
import jax
import jax.numpy as jnp
from jax.experimental import pallas as pl
from jax.experimental.pallas import tpu as pltpu

_SMOOTH = 1e-06


def _dice_body(nH, B, C, pred_ref, tgt_ref, out_ref, acc_i, acc_s, acc_c,
               dsum_ref):
    b = pl.program_id(0)
    h = pl.program_id(1)

    @pl.when(h == 0)
    def _init():
        acc_i[...] = jnp.zeros_like(acc_i)
        acc_s[...] = jnp.zeros_like(acc_s)
        acc_c[...] = jnp.zeros_like(acc_c)

    @pl.when((b == 0) & (h == 0))
    def _init_scalar():
        dsum_ref[0] = 0.0

    p = pred_ref[0]
    t = tgt_ref[0]
    Hb = t.shape[0]
    for k in range(Hb // 8):
        tk = t[k * 8:(k + 1) * 8, :]
        for c in range(C):
            pk = p[c, k * 8:(k + 1) * 8, :]
            m = tk == c
            acc_i[c, :, :] += jnp.where(m, pk, 0.0)
            acc_c[c, :, :] += jnp.where(m, 1.0, 0.0)
            acc_s[c, :, :] += pk

    @pl.when(h == nH - 1)
    def _finish_batch():
        total = dsum_ref[0]
        for c in range(C):
            inter = jnp.sum(acc_i[c, :, :])
            psum = jnp.sum(acc_s[c, :, :])
            cnt = jnp.sum(acc_c[c, :, :])
            total += (2.0 * inter + _SMOOTH) / (psum + cnt + _SMOOTH)
        dsum_ref[0] = total

    @pl.when((h == nH - 1) & (b == B - 1))
    def _emit():
        out_ref[0] = 1.0 - dsum_ref[0] / (B * C)


def kernel(pred, target):
    B, C, H, W = pred.shape
    Hb = 32
    nH = H // Hb

    import functools
    body = functools.partial(_dice_body, nH, B, C)

    out = pl.pallas_call(
        body,
        grid=(B, nH),
        in_specs=[
            pl.BlockSpec((1, C, Hb, W), lambda b, h: (b, 0, h, 0)),
            pl.BlockSpec((1, Hb, W), lambda b, h: (b, h, 0)),
        ],
        out_specs=pl.BlockSpec(memory_space=pltpu.SMEM),
        out_shape=jax.ShapeDtypeStruct((1,), jnp.float32),
        scratch_shapes=[
            pltpu.VMEM((C, 8, W), jnp.float32),
            pltpu.VMEM((C, 8, W), jnp.float32),
            pltpu.VMEM((C, 8, W), jnp.float32),
            pltpu.SMEM((1,), jnp.float32),
        ],
        compiler_params=pltpu.CompilerParams(
            dimension_semantics=("arbitrary", "arbitrary")),
    )(pred, target)
    return out[0]


# speedup vs baseline: 111.4302x; 1.4090x over previous
import jax
import jax.numpy as jnp
from jax.experimental import pallas as pl
from jax.experimental.pallas import tpu as pltpu

_SMOOTH = 1e-06


def _dice_body(nH, B, C, pred_ref, tgt_ref, out_ref, acc_i, acc_s, acc_c,
               dsum_ref):
    b = pl.program_id(0)
    h = pl.program_id(1)

    @pl.when(h == 0)
    def _init():
        acc_i[...] = jnp.zeros_like(acc_i)
        acc_s[...] = jnp.zeros_like(acc_s)
        acc_c[...] = jnp.zeros_like(acc_c)

    @pl.when((b == 0) & (h == 0))
    def _init_scalar():
        dsum_ref[0] = 0.0

    p = pred_ref[0]
    t = tgt_ref[0]
    Hb = t.shape[0]
    for c in range(C):
        ai = acc_i[c, :, :]
        ac = acc_c[c, :, :]
        asum = acc_s[c, :, :]
        for k in range(Hb // 8):
            tk = t[k * 8:(k + 1) * 8, :]
            pk = p[c, k * 8:(k + 1) * 8, :]
            m = tk == c
            ai = ai + jnp.where(m, pk, 0.0)
            ac = ac + jnp.where(m, 1.0, 0.0)
            asum = asum + pk
        acc_i[c, :, :] = ai
        acc_c[c, :, :] = ac
        acc_s[c, :, :] = asum

    @pl.when(h == nH - 1)
    def _finish_batch():
        total = dsum_ref[0]
        for c in range(C):
            inter = jnp.sum(acc_i[c, :, :])
            psum = jnp.sum(acc_s[c, :, :])
            cnt = jnp.sum(acc_c[c, :, :])
            total += (2.0 * inter + _SMOOTH) / (psum + cnt + _SMOOTH)
        dsum_ref[0] = total

    @pl.when((h == nH - 1) & (b == B - 1))
    def _emit():
        out_ref[0] = 1.0 - dsum_ref[0] / (B * C)


def kernel(pred, target):
    B, C, H, W = pred.shape
    Hb = 64
    nH = H // Hb

    import functools
    body = functools.partial(_dice_body, nH, B, C)

    out = pl.pallas_call(
        body,
        grid=(B, nH),
        in_specs=[
            pl.BlockSpec((1, C, Hb, W), lambda b, h: (b, 0, h, 0)),
            pl.BlockSpec((1, Hb, W), lambda b, h: (b, h, 0)),
        ],
        out_specs=pl.BlockSpec(memory_space=pltpu.SMEM),
        out_shape=jax.ShapeDtypeStruct((1,), jnp.float32),
        scratch_shapes=[
            pltpu.VMEM((C, 8, W), jnp.float32),
            pltpu.VMEM((C, 8, W), jnp.float32),
            pltpu.VMEM((C, 8, W), jnp.float32),
            pltpu.SMEM((1,), jnp.float32),
        ],
        compiler_params=pltpu.CompilerParams(
            dimension_semantics=("arbitrary", "arbitrary")),
    )(pred, target)
    return out[0]


# speedup vs baseline: 112.0137x; 1.0052x over previous
import jax
import jax.numpy as jnp
from jax.experimental import pallas as pl
from jax.experimental.pallas import tpu as pltpu

_SMOOTH = 1e-06


def _dice_body(nH, B, C, pred_ref, tgt_ref, out_ref, acc_i, acc_s, acc_c,
               dsum_ref):
    b = pl.program_id(0)
    h = pl.program_id(1)

    @pl.when(h == 0)
    def _init():
        acc_i[...] = jnp.zeros_like(acc_i)
        acc_s[...] = jnp.zeros_like(acc_s)
        acc_c[...] = jnp.zeros_like(acc_c)

    @pl.when((b == 0) & (h == 0))
    def _init_scalar():
        dsum_ref[0] = 0.0

    Hb = tgt_ref.shape[1]
    for c in range(C):
        ai = acc_i[c, :, :]
        ac = acc_c[c, :, :]
        asum = acc_s[c, :, :]
        for k in range(Hb // 8):
            tk = tgt_ref[0, k * 8:(k + 1) * 8, :]
            pk = pred_ref[0, c, k * 8:(k + 1) * 8, :]
            m = tk == c
            ai = ai + jnp.where(m, pk, 0.0)
            ac = ac + jnp.where(m, 1.0, 0.0)
            asum = asum + pk
        acc_i[c, :, :] = ai
        acc_c[c, :, :] = ac
        acc_s[c, :, :] = asum

    @pl.when(h == nH - 1)
    def _finish_batch():
        total = dsum_ref[0]
        for c in range(C):
            inter = jnp.sum(acc_i[c, :, :])
            psum = jnp.sum(acc_s[c, :, :])
            cnt = jnp.sum(acc_c[c, :, :])
            total += (2.0 * inter + _SMOOTH) / (psum + cnt + _SMOOTH)
        dsum_ref[0] = total

    @pl.when((h == nH - 1) & (b == B - 1))
    def _emit():
        out_ref[0] = 1.0 - dsum_ref[0] / (B * C)


def kernel(pred, target):
    B, C, H, W = pred.shape
    Hb = 64
    nH = H // Hb

    import functools
    body = functools.partial(_dice_body, nH, B, C)

    out = pl.pallas_call(
        body,
        grid=(B, nH),
        in_specs=[
            pl.BlockSpec((1, C, Hb, W), lambda b, h: (b, 0, h, 0)),
            pl.BlockSpec((1, Hb, W), lambda b, h: (b, h, 0)),
        ],
        out_specs=pl.BlockSpec(memory_space=pltpu.SMEM),
        out_shape=jax.ShapeDtypeStruct((1,), jnp.float32),
        scratch_shapes=[
            pltpu.VMEM((C, 8, W), jnp.float32),
            pltpu.VMEM((C, 8, W), jnp.float32),
            pltpu.VMEM((C, 8, W), jnp.float32),
            pltpu.SMEM((1,), jnp.float32),
        ],
        compiler_params=pltpu.CompilerParams(
            dimension_semantics=("arbitrary", "arbitrary")),
    )(pred, target)
    return out[0]


# speedup vs baseline: 138.0943x; 1.2328x over previous
import jax
import jax.numpy as jnp
from jax.experimental import pallas as pl
from jax.experimental.pallas import tpu as pltpu

_SMOOTH = 1e-06


def _dice_body(nH, B, C, pred_ref, tgt_ref, out_ref, acc_i, acc_s, acc_c,
               dsum_ref):
    b = pl.program_id(0)
    h = pl.program_id(1)

    @pl.when(h == 0)
    def _init():
        acc_i[...] = jnp.zeros_like(acc_i)
        acc_s[...] = jnp.zeros_like(acc_s)
        acc_c[...] = jnp.zeros_like(acc_c)

    @pl.when((b == 0) & (h == 0))
    def _init_scalar():
        dsum_ref[0] = 0.0

    Hb = tgt_ref.shape[1]

    def _tree128(x):
        return (x[:, 0:128] + x[:, 128:256]) + (x[:, 256:384] + x[:, 384:512])

    for c in range(C):
        ai = acc_i[c, :, :]
        ac = acc_c[c, :, :]
        asum = acc_s[c, :, :]
        for k in range(Hb // 8):
            tk = tgt_ref[0, k * 8:(k + 1) * 8, :]
            pk = pred_ref[0, c, k * 8:(k + 1) * 8, :]
            m = tk == c
            ai = ai + _tree128(jnp.where(m, pk, 0.0))
            ac = ac + _tree128(jnp.where(m, 1.0, 0.0))
            asum = asum + _tree128(pk)
        acc_i[c, :, :] = ai
        acc_c[c, :, :] = ac
        acc_s[c, :, :] = asum

    @pl.when(h == nH - 1)
    def _finish_batch():
        total = dsum_ref[0]
        for c in range(C):
            inter = jnp.sum(acc_i[c, :, :])
            psum = jnp.sum(acc_s[c, :, :])
            cnt = jnp.sum(acc_c[c, :, :])
            total += (2.0 * inter + _SMOOTH) / (psum + cnt + _SMOOTH)
        dsum_ref[0] = total

    @pl.when((h == nH - 1) & (b == B - 1))
    def _emit():
        out_ref[0] = 1.0 - dsum_ref[0] / (B * C)


def kernel(pred, target):
    B, C, H, W = pred.shape
    Hb = 128
    nH = H // Hb

    import functools
    body = functools.partial(_dice_body, nH, B, C)

    out = pl.pallas_call(
        body,
        grid=(B, nH),
        in_specs=[
            pl.BlockSpec((1, C, Hb, W), lambda b, h: (b, 0, h, 0)),
            pl.BlockSpec((1, Hb, W), lambda b, h: (b, h, 0)),
        ],
        out_specs=pl.BlockSpec(memory_space=pltpu.SMEM),
        out_shape=jax.ShapeDtypeStruct((1,), jnp.float32),
        scratch_shapes=[
            pltpu.VMEM((C, 8, 128), jnp.float32),
            pltpu.VMEM((C, 8, 128), jnp.float32),
            pltpu.VMEM((C, 8, 128), jnp.float32),
            pltpu.SMEM((1,), jnp.float32),
        ],
        compiler_params=pltpu.CompilerParams(
            dimension_semantics=("arbitrary", "arbitrary")),
    )(pred, target)
    return out[0]


# speedup vs baseline: 153.8485x; 1.1141x over previous
import jax
import jax.numpy as jnp
from jax.experimental import pallas as pl
from jax.experimental.pallas import tpu as pltpu

_SMOOTH = 1e-06


def _dice_body(nH, B, C, pred_ref, tgt_ref, out_ref, acc_i, acc_s, acc_c,
               dsum_ref):
    b = pl.program_id(0)
    h = pl.program_id(1)

    @pl.when(h == 0)
    def _init():
        acc_i[...] = jnp.zeros_like(acc_i)
        acc_s[...] = jnp.zeros_like(acc_s)
        acc_c[...] = jnp.zeros_like(acc_c)

    @pl.when((b == 0) & (h == 0))
    def _init_scalar():
        dsum_ref[0] = 0.0

    Hb = tgt_ref.shape[1]

    def _tree128(x):
        return (x[:, 0:128] + x[:, 128:256]) + (x[:, 256:384] + x[:, 384:512])

    for c in range(C):
        ai = acc_i[c, :, :]
        ac = acc_c[c, :, :]
        asum = acc_s[c, :, :]
        for k in range(Hb // 8):
            tk = tgt_ref[0, k * 8:(k + 1) * 8, :]
            pk = pred_ref[0, c, k * 8:(k + 1) * 8, :]
            m = tk == c
            ai = ai + _tree128(jnp.where(m, pk, 0.0))
            ac = ac + _tree128(jnp.where(m, 1.0, 0.0))
            asum = asum + _tree128(pk)
        acc_i[c, :, :] = ai
        acc_c[c, :, :] = ac
        acc_s[c, :, :] = asum

    @pl.when(h == nH - 1)
    def _finish_batch():
        total = dsum_ref[0]
        for c in range(C):
            inter = jnp.sum(acc_i[c, :, :])
            psum = jnp.sum(acc_s[c, :, :])
            cnt = jnp.sum(acc_c[c, :, :])
            total += (2.0 * inter + _SMOOTH) / (psum + cnt + _SMOOTH)
        dsum_ref[0] = total

    @pl.when((h == nH - 1) & (b == B - 1))
    def _emit():
        out_ref[0] = 1.0 - dsum_ref[0] / (B * C)


def kernel(pred, target):
    B, C, H, W = pred.shape
    Hb = 256
    nH = H // Hb

    import functools
    body = functools.partial(_dice_body, nH, B, C)

    out = pl.pallas_call(
        body,
        grid=(B, nH),
        in_specs=[
            pl.BlockSpec((1, C, Hb, W), lambda b, h: (b, 0, h, 0)),
            pl.BlockSpec((1, Hb, W), lambda b, h: (b, h, 0)),
        ],
        out_specs=pl.BlockSpec(memory_space=pltpu.SMEM),
        out_shape=jax.ShapeDtypeStruct((1,), jnp.float32),
        scratch_shapes=[
            pltpu.VMEM((C, 8, 128), jnp.float32),
            pltpu.VMEM((C, 8, 128), jnp.float32),
            pltpu.VMEM((C, 8, 128), jnp.float32),
            pltpu.SMEM((1,), jnp.float32),
        ],
        compiler_params=pltpu.CompilerParams(
            dimension_semantics=("arbitrary", "arbitrary")),
    )(pred, target)
    return out[0]


# speedup vs baseline: 176.4038x; 1.1466x over previous
import jax
import jax.numpy as jnp
from jax.experimental import pallas as pl
from jax.experimental.pallas import tpu as pltpu

_SMOOTH = 1e-06


def _dice_body(nH, B, C, pred_ref, tgt_ref, out_ref, acc_i, acc_s, acc_c,
               dsum_ref):
    b = pl.program_id(0)
    h = pl.program_id(1)

    @pl.when(h == 0)
    def _init():
        acc_i[...] = jnp.zeros_like(acc_i)
        acc_s[...] = jnp.zeros_like(acc_s)
        acc_c[...] = jnp.zeros_like(acc_c)

    @pl.when((b == 0) & (h == 0))
    def _init_scalar():
        dsum_ref[0] = 0.0

    Hb = tgt_ref.shape[1]

    def _tree128(x):
        return (x[:, 0:128] + x[:, 128:256]) + (x[:, 256:384] + x[:, 384:512])

    for c in range(C):
        ai = acc_i[c, :, :]
        ac = acc_c[c, :, :]
        asum = acc_s[c, :, :]
        for k in range(Hb // 8):
            tk = tgt_ref[0, k * 8:(k + 1) * 8, :]
            pk = pred_ref[0, c, k * 8:(k + 1) * 8, :]
            m = tk == c
            ai = ai + _tree128(jnp.where(m, pk, 0.0))
            ac = ac + _tree128(jnp.where(m, 1.0, 0.0))
            asum = asum + _tree128(pk)
        acc_i[c, :, :] = ai
        acc_c[c, :, :] = ac
        acc_s[c, :, :] = asum

    @pl.when(h == nH - 1)
    def _finish_batch():
        total = dsum_ref[0]
        for c in range(C):
            inter = jnp.sum(acc_i[c, :, :])
            psum = jnp.sum(acc_s[c, :, :])
            cnt = jnp.sum(acc_c[c, :, :])
            total += (2.0 * inter + _SMOOTH) / (psum + cnt + _SMOOTH)
        dsum_ref[0] = total

    @pl.when((h == nH - 1) & (b == B - 1))
    def _emit():
        out_ref[0] = 1.0 - dsum_ref[0] / (B * C)


def kernel(pred, target):
    B, C, H, W = pred.shape
    Hb = 512
    nH = H // Hb

    import functools
    body = functools.partial(_dice_body, nH, B, C)

    out = pl.pallas_call(
        body,
        grid=(B, nH),
        in_specs=[
            pl.BlockSpec((1, C, Hb, W), lambda b, h: (b, 0, h, 0)),
            pl.BlockSpec((1, Hb, W), lambda b, h: (b, h, 0)),
        ],
        out_specs=pl.BlockSpec(memory_space=pltpu.SMEM),
        out_shape=jax.ShapeDtypeStruct((1,), jnp.float32),
        scratch_shapes=[
            pltpu.VMEM((C, 8, 128), jnp.float32),
            pltpu.VMEM((C, 8, 128), jnp.float32),
            pltpu.VMEM((C, 8, 128), jnp.float32),
            pltpu.SMEM((1,), jnp.float32),
        ],
        compiler_params=pltpu.CompilerParams(
            dimension_semantics=("arbitrary", "arbitrary")),
    )(pred, target)
    return out[0]
